# Initial kernel scaffold; baseline (speedup 1.0000x reference)
import jax
import jax.numpy as jnp
from jax.experimental import pallas as pl


def kernel(feats, mapping):
    raise NotImplementedError("write your pallas kernel here")


# speedup vs baseline: 20.4906x; 20.4906x over previous
import functools

import jax
import jax.numpy as jnp
from jax import lax
from jax.experimental import pallas as pl
from jax.experimental.pallas import tpu as pltpu
from jax.experimental.pallas import tpu_sc as plsc

B, C, H, W = 2, 64, 184, 180
HW = H * W
D = B * C
NPTS = 16384
NV = 32

_info = plsc.get_sparse_core_info()
NC, NS, L = _info.num_cores, _info.num_subcores, _info.num_lanes
NW = NC * NS
PW = NPTS // NW
NPC = 4
NCHUNK = PW // NPC
IDX_PER_CHUNK = NPC * NV


def _sc_gather_mean(table, idx):
    mesh = plsc.VectorSubcoreMesh(core_axis_name="c", subcore_axis_name="s")

    @functools.partial(
        pl.kernel,
        mesh=mesh,
        out_type=jax.ShapeDtypeStruct((NPTS, D), jnp.float32),
        scratch_types=[
            pltpu.VMEM((NCHUNK, IDX_PER_CHUNK), jnp.int32),
            pltpu.VMEM((IDX_PER_CHUNK, D), jnp.float32),
            pltpu.VMEM((NPC, D), jnp.float32),
            pltpu.SemaphoreType.DMA,
        ],
    )
    def k(table_hbm, idx_hbm, out_hbm, idx_v, rows_v, stage_v, sem):
        wid = lax.axis_index("s") * NC + lax.axis_index("c")
        pltpu.sync_copy(idx_hbm.at[wid], idx_v)

        def chunk_body(ci, _):
            pltpu.async_copy(table_hbm.at[idx_v.at[ci]], rows_v, sem).wait()

            def pt_body(jc, _):
                j = jc // (D // L)
                col = (jc % (D // L)) * L
                base = j * NV
                acc = rows_v[base, pl.ds(col, L)]
                for r in range(1, NV):
                    acc = acc + rows_v[base + r, pl.ds(col, L)]
                stage_v[j, pl.ds(col, L)] = acc * (1.0 / NV)
                return 0

            lax.fori_loop(0, NPC * (D // L), pt_body, 0)
            pltpu.sync_copy(stage_v, out_hbm.at[pl.ds(wid * PW + ci * NPC, NPC)])
            return 0

        lax.fori_loop(0, NCHUNK, chunk_body, 0)

    return k(table, idx)


def kernel(feats, mapping):
    table = jnp.transpose(feats.reshape(D, HW))
    idx = mapping.reshape(NW, NCHUNK, IDX_PER_CHUNK)
    out_rows = _sc_gather_mean(table, idx)
    return jnp.transpose(out_rows).reshape(B, C, NPTS, 1)


# speedup vs baseline: 30.1409x; 1.4710x over previous
import functools

import jax
import jax.numpy as jnp
from jax import lax
from jax.experimental import pallas as pl
from jax.experimental.pallas import tpu as pltpu
from jax.experimental.pallas import tpu_sc as plsc

B, C, H, W = 2, 64, 184, 180
HW = H * W
D = B * C
NPTS = 16384
NV = 32

_info = plsc.get_sparse_core_info()
NC, NS, L = _info.num_cores, _info.num_subcores, _info.num_lanes
NW = NC * NS
PW = NPTS // NW
NPC = 4
NCHUNK = PW // NPC
IDX_PER_CHUNK = NPC * NV
CG = D // L


def _sc_gather_mean(table, idx):
    mesh = plsc.VectorSubcoreMesh(core_axis_name="c", subcore_axis_name="s")

    @functools.partial(
        pl.kernel,
        mesh=mesh,
        out_type=jax.ShapeDtypeStruct((NPTS, D), jnp.float32),
        scratch_types=[
            pltpu.VMEM((NCHUNK, IDX_PER_CHUNK), jnp.int32),
            pltpu.VMEM((2, IDX_PER_CHUNK, D), jnp.float32),
            pltpu.VMEM((PW, D), jnp.float32),
            pltpu.SemaphoreType.DMA,
            pltpu.SemaphoreType.DMA,
        ],
    )
    def k(table_hbm, idx_hbm, out_hbm, idx_v, rows_v, outst_v, sem0, sem1):
        wid = lax.axis_index("s") * NC + lax.axis_index("c")
        pltpu.sync_copy(idx_hbm.at[wid], idx_v)
        sems = (sem0, sem1)

        def gather(ci, b, sem):
            return pltpu.make_async_copy(
                table_hbm.at[idx_v.at[ci]], rows_v.at[b], sem)

        gather(0, 0, sem0).start()
        gather(1, 1, sem1).start()

        def pair_body(g, _):
            for b in range(2):
                ci = g * 2 + b
                gather(ci, b, sems[b]).wait()

                def pt_body(j, _):
                    base = j * NV
                    row = ci * NPC + j
                    for cg in range(CG):
                        col = cg * L
                        acc = rows_v[b, base, pl.ds(col, L)]
                        for r in range(1, NV):
                            acc = acc + rows_v[b, base + r, pl.ds(col, L)]
                        outst_v[row, pl.ds(col, L)] = acc
                    return 0

                lax.fori_loop(0, NPC, pt_body, 0)

                @pl.when(ci + 2 < NCHUNK)
                def _():
                    gather(ci + 2, b, sems[b]).start()

            return 0

        lax.fori_loop(0, NCHUNK // 2, pair_body, 0)
        pltpu.sync_copy(outst_v, out_hbm.at[pl.ds(wid * PW, PW)])

    return k(table, idx)


def kernel(feats, mapping):
    table = jnp.transpose((feats * (1.0 / NV)).reshape(D, HW))
    idx = mapping.reshape(NW, NCHUNK, IDX_PER_CHUNK)
    out_rows = _sc_gather_mean(table, idx)
    return jnp.transpose(out_rows).reshape(B, C, NPTS, 1)
